# baseline (speedup 1.0000x reference)
import jax
import jax.numpy as jnp
from jax import lax
from jax.experimental import pallas as pl
from jax.experimental.pallas import tpu as pltpu
from jax.experimental.pallas import tpu_sc as plsc

_BITS = 8
_EBIT = 8
_L = 16
_NW = 32
_CH = 2
_RND = 12582912.0


def _row_quantize(src, dst, r, nvec):
    def maxbody(it, accs):
        a = list(accs)
        for k in range(8):
            sl = pl.ds(pl.multiple_of((it * 8 + k) * _L, _L), _L)
            a[k % 4] = jnp.maximum(a[k % 4], jnp.abs(src[r, sl]))
        return tuple(a)

    init = jnp.full((_L,), 1e-10, jnp.float32)
    a0, a1, a2, a3 = lax.fori_loop(0, nvec // 8, maxbody, (init,) * 4)
    m = jnp.maximum(jnp.maximum(a0, a1), jnp.maximum(a2, a3))
    idx = lax.iota(jnp.int32, _L)
    for sh in (1, 2, 4, 8):
        m = jnp.maximum(m, m[jnp.bitwise_xor(idx, sh)])
    ebits = lax.shift_right_logical(lax.bitcast_convert_type(m, jnp.int32), 23)
    e = jnp.clip(ebits - 127, -(2 ** (_EBIT - 1)), 2 ** (_EBIT - 1) - 1)
    scale = lax.bitcast_convert_type(
        lax.shift_left(((_BITS - 2) - e) + 127, 23), jnp.float32)
    iscale = lax.bitcast_convert_type(
        lax.shift_left((e - (_BITS - 2)) + 127, 23), jnp.float32)

    @plsc.parallel_loop(0, nvec, step=1, unroll=8)
    def _(i):
        sl = pl.ds(pl.multiple_of(i * _L, _L), _L)
        v = src[r, sl]
        d = jnp.where(v >= 0, jnp.maximum(v, 1e-10), jnp.minimum(v, -1e-10))
        q = (d * scale + _RND) - _RND
        q = jnp.clip(q, -(2.0 ** (_BITS - 1)), 2.0 ** (_BITS - 1) - 1)
        dst[r, sl] = q * iscale


def _sc_body(x_hbm, o_hbm, in0, in1, out0, out1, si0, si1, so0, so1):
    n = x_hbm.shape[1]
    nvec = n // _L
    rows_per_w = x_hbm.shape[0] // _NW
    nch = rows_per_w // _CH
    half = nch // 2
    wid = lax.axis_index("s") * 2 + lax.axis_index("c")
    base = wid * rows_per_w

    ins, outs = (in0, in1), (out0, out1)
    sis, sos = (si0, si1), (so0, so1)

    def in_slice(ch):
        return x_hbm.at[pl.ds(base + ch * _CH, _CH)]

    def out_slice(ch):
        return o_hbm.at[pl.ds(base + ch * _CH, _CH)]

    pltpu.async_copy(in_slice(0), in0, si0)
    pltpu.async_copy(in_slice(1), in1, si1)

    def outer(o, _):
        for b in range(2):
            ch = o * 2 + b
            pltpu.make_async_copy(in_slice(ch), ins[b], sis[b]).wait()

            @pl.when(o > 0)
            def _():
                pltpu.make_async_copy(outs[b], out_slice(ch - 2), sos[b]).wait()

            for r in range(_CH):
                _row_quantize(ins[b], outs[b], r, nvec)
            pltpu.async_copy(outs[b], out_slice(ch), sos[b])

            @pl.when(o + 1 < half)
            def _():
                pltpu.async_copy(in_slice(ch + 2), ins[b], sis[b])
        return 0

    lax.fori_loop(0, half, outer, 0)
    pltpu.make_async_copy(out0, out_slice(nch - 2), so0).wait()
    pltpu.make_async_copy(out1, out_slice(nch - 1), so1).wait()


def kernel(x):
    B, N = x.shape
    mesh = plsc.VectorSubcoreMesh(core_axis_name="c", subcore_axis_name="s")
    f = pl.kernel(
        _sc_body,
        out_type=jax.ShapeDtypeStruct((B, N), x.dtype),
        mesh=mesh,
        scratch_types=[
            pltpu.VMEM((_CH, N), jnp.float32),
            pltpu.VMEM((_CH, N), jnp.float32),
            pltpu.VMEM((_CH, N), jnp.float32),
            pltpu.VMEM((_CH, N), jnp.float32),
            pltpu.SemaphoreType.DMA,
            pltpu.SemaphoreType.DMA,
            pltpu.SemaphoreType.DMA,
            pltpu.SemaphoreType.DMA,
        ],
    )
    return f(x)


# speedup vs baseline: 1.3920x; 1.3920x over previous
import jax
import jax.numpy as jnp
from jax import lax
from jax.experimental import pallas as pl
from jax.experimental.pallas import tpu as pltpu
from jax.experimental.pallas import tpu_sc as plsc

_BITS = 8
_EBIT = 8
_L = 16
_NW = 32
_CH = 2
_RND = 12582912.0


def _row_quantize(src, dst, r, nvec):
    def maxbody(it, accs):
        a = list(accs)
        for k in range(8):
            sl = pl.ds(pl.multiple_of((it * 8 + k) * _L, _L), _L)
            a[k % 4] = jnp.maximum(a[k % 4], jnp.abs(src[r, sl]))
        return tuple(a)

    init = jnp.full((_L,), 1e-10, jnp.float32)
    a0, a1, a2, a3 = lax.fori_loop(0, nvec // 8, maxbody, (init,) * 4)
    m = jnp.maximum(jnp.maximum(a0, a1), jnp.maximum(a2, a3))
    idx = lax.iota(jnp.int32, _L)
    for sh in (1, 2, 4, 8):
        m = jnp.maximum(m, m[jnp.bitwise_xor(idx, sh)])
    ebits = lax.shift_right_logical(lax.bitcast_convert_type(m, jnp.int32), 23)
    e = jnp.clip(ebits - 127, -(2 ** (_EBIT - 1)), 2 ** (_EBIT - 1) - 1)
    scale = lax.bitcast_convert_type(
        lax.shift_left(((_BITS - 2) - e) + 127, 23), jnp.float32)
    iscale = lax.bitcast_convert_type(
        lax.shift_left((e - (_BITS - 2)) + 127, 23), jnp.float32)

    @pl.when(e[0] >= -26)
    def _():
        @plsc.parallel_loop(0, nvec, step=1, unroll=8)
        def _(i):
            sl = pl.ds(pl.multiple_of(i * _L, _L), _L)
            q = (src[r, sl] * scale + _RND) - _RND
            q = jnp.minimum(q, 2.0 ** (_BITS - 1) - 1)
            dst[r, sl] = q * iscale

    @pl.when(e[0] < -26)
    def _():
        @plsc.parallel_loop(0, nvec, step=1, unroll=8)
        def _(i):
            sl = pl.ds(pl.multiple_of(i * _L, _L), _L)
            v = src[r, sl]
            d = jnp.where(v >= 0, jnp.maximum(v, 1e-10),
                          jnp.minimum(v, -1e-10))
            q = (d * scale + _RND) - _RND
            q = jnp.clip(q, -(2.0 ** (_BITS - 1)), 2.0 ** (_BITS - 1) - 1)
            dst[r, sl] = q * iscale


def _sc_body(x_hbm, o_hbm, in0, in1, out0, out1, si0, si1, so0, so1):
    n = x_hbm.shape[1]
    nvec = n // _L
    rows_per_w = x_hbm.shape[0] // _NW
    nch = rows_per_w // _CH
    half = nch // 2
    wid = lax.axis_index("s") * 2 + lax.axis_index("c")
    base = wid * rows_per_w

    ins, outs = (in0, in1), (out0, out1)
    sis, sos = (si0, si1), (so0, so1)

    def in_slice(ch):
        return x_hbm.at[pl.ds(base + ch * _CH, _CH)]

    def out_slice(ch):
        return o_hbm.at[pl.ds(base + ch * _CH, _CH)]

    pltpu.async_copy(in_slice(0), in0, si0)
    pltpu.async_copy(in_slice(1), in1, si1)

    def outer(o, _):
        for b in range(2):
            ch = o * 2 + b
            pltpu.make_async_copy(in_slice(ch), ins[b], sis[b]).wait()

            @pl.when(o > 0)
            def _():
                pltpu.make_async_copy(outs[b], out_slice(ch - 2), sos[b]).wait()

            for r in range(_CH):
                _row_quantize(ins[b], outs[b], r, nvec)
            pltpu.async_copy(outs[b], out_slice(ch), sos[b])

            @pl.when(o + 1 < half)
            def _():
                pltpu.async_copy(in_slice(ch + 2), ins[b], sis[b])
        return 0

    lax.fori_loop(0, half, outer, 0)
    pltpu.make_async_copy(out0, out_slice(nch - 2), so0).wait()
    pltpu.make_async_copy(out1, out_slice(nch - 1), so1).wait()


def kernel(x):
    B, N = x.shape
    mesh = plsc.VectorSubcoreMesh(core_axis_name="c", subcore_axis_name="s")
    f = pl.kernel(
        _sc_body,
        out_type=jax.ShapeDtypeStruct((B, N), x.dtype),
        mesh=mesh,
        scratch_types=[
            pltpu.VMEM((_CH, N), jnp.float32),
            pltpu.VMEM((_CH, N), jnp.float32),
            pltpu.VMEM((_CH, N), jnp.float32),
            pltpu.VMEM((_CH, N), jnp.float32),
            pltpu.SemaphoreType.DMA,
            pltpu.SemaphoreType.DMA,
            pltpu.SemaphoreType.DMA,
            pltpu.SemaphoreType.DMA,
        ],
    )
    return f(x)
